# baseline (speedup 1.0000x reference)
import functools

import jax
import jax.numpy as jnp
from jax import lax
from jax.experimental import pallas as pl
from jax.experimental.pallas import tpu as pltpu
from jax.experimental.pallas import tpu_sc as plsc


def _sort_body(v_ref, o_ref):
    x = v_ref[0]
    L = x.shape[0]
    ii = lax.broadcasted_iota(jnp.int32, (L, 1), 0)
    nbits = L.bit_length() - 1

    def phase(p, x):
        dirmask = 2 << p

        def step(t, x):
            d = 1 << (p - t)
            up = pltpu.roll(x, L - d, axis=0)
            dn = pltpu.roll(x, d, axis=0)
            is_lower = (ii & d) == 0
            part = jnp.where(is_lower, up, dn)
            asc = (ii & dirmask) == 0
            take_min = asc == is_lower
            return jnp.where(take_min, jnp.minimum(x, part),
                             jnp.maximum(x, part))

        return lax.fori_loop(0, p + 1, step, x)

    o_ref[0] = lax.fori_loop(0, nbits, phase, x)


def _sort_columns(v):
    B, L, D = v.shape
    CB = min(128, D)
    return pl.pallas_call(
        _sort_body,
        grid=(B, D // CB),
        in_specs=[pl.BlockSpec((1, L, CB), lambda b, c: (b, 0, c))],
        out_specs=pl.BlockSpec((1, L, CB), lambda b, c: (b, 0, c)),
        out_shape=jax.ShapeDtypeStruct((B, L, D), jnp.float32),
    )(v)


def _rank_body(L, cT_ref, ci_ref, rank_ref, cls_ref, acc):
    b = pl.program_id(0)
    i_blk = pl.program_id(1)
    CI = ci_ref.shape[-1]
    cT = cT_ref[...]
    bsel = lax.broadcasted_iota(jnp.int32, cT.shape, 1) == b
    vj = jnp.sum(jnp.where(bsel, cT, 0.0), axis=1, keepdims=True)
    vi = ci_ref[0]
    jj = lax.broadcasted_iota(jnp.int32, (L, 1), 0)
    iidx = i_blk * CI + lax.broadcasted_iota(jnp.int32, (1, CI), 1)
    lt = vj < vi
    eq = (vj == vi) & (jj < iidx)
    cnt = jnp.sum((lt | eq).astype(jnp.int32), axis=0, keepdims=True)
    rank_ref[0] = cnt + b * L

    @pl.when(i_blk == 0)
    def _():
        acc[...] = jnp.zeros_like(acc)

    acc[...] += jnp.where(cnt == 0, iidx, 0)

    @pl.when(i_blk == pl.num_programs(1) - 1)
    def _():
        cls_ref[0] = jnp.broadcast_to(
            jnp.sum(acc[...], axis=1, keepdims=True), cls_ref.shape[1:])


def _rank_col0(col0):
    B, L = col0.shape
    CI = min(128, L)
    NI = L // CI
    col0T = col0.T.reshape(L, B)
    col0i = col0.reshape(B * NI, 1, CI)
    frank, cls3 = pl.pallas_call(
        functools.partial(_rank_body, L),
        grid=(B, NI),
        in_specs=[
            pl.BlockSpec((L, B), lambda b, i: (0, 0)),
            pl.BlockSpec((1, 1, CI), lambda b, i: (b * NI + i, 0, 0)),
        ],
        out_specs=[
            pl.BlockSpec((1, 1, CI), lambda b, i: (b * NI + i, 0, 0)),
            pl.BlockSpec((1, 1, CI), lambda b, i: (b, 0, 0)),
        ],
        out_shape=[
            jax.ShapeDtypeStruct((B * NI, 1, CI), jnp.int32),
            jax.ShapeDtypeStruct((B, 1, CI), jnp.int32),
        ],
        scratch_shapes=[pltpu.VMEM((1, CI), jnp.int32)],
    )(col0T, col0i)
    return frank.reshape(B * L), cls3[:, 0, :1]


def _make_gather(R, D):
    info = plsc.get_sparse_core_info()
    NW = info.num_cores * info.num_subcores
    rows_w = R // NW
    CH = 64
    NCH = rows_w // CH
    mesh = plsc.VectorSubcoreMesh(core_axis_name="c", subcore_axis_name="s")

    @functools.partial(
        pl.kernel,
        out_type=jax.ShapeDtypeStruct((R, D), jnp.float32),
        mesh=mesh,
        scratch_types=[
            pltpu.VMEM((CH,), jnp.int32),
            pltpu.VMEM((CH, D), jnp.float32),
            pltpu.SemaphoreType.DMA,
        ],
    )
    def gather(table_hbm, idx_hbm, out_hbm, idx_v, rows_v, sem):
        wid = lax.axis_index("s") * info.num_cores + lax.axis_index("c")
        base = wid * rows_w

        def chunk(c, carry):
            off = base + c * CH
            pltpu.sync_copy(idx_hbm.at[pl.ds(off, CH)], idx_v)
            pltpu.async_copy(table_hbm.at[idx_v], rows_v, sem).wait()
            pltpu.sync_copy(rows_v, out_hbm.at[pl.ds(off, CH)])
            return carry

        lax.fori_loop(0, NCH, chunk, 0)

    return gather


def kernel(q, k, v):
    del q, k
    B, L, D = v.shape
    frank, cls = _rank_col0(v[:, :, 0])
    sorted_v = _sort_columns(v)
    out = _make_gather(B * L, D)(sorted_v.reshape(B * L, D), frank)
    out = out.reshape(B, L, D)
    return (out, out, cls)


# speedup vs baseline: 3.5638x; 3.5638x over previous
import functools

import jax
import jax.numpy as jnp
from jax import lax
from jax.experimental import pallas as pl
from jax.experimental.pallas import tpu as pltpu
from jax.experimental.pallas import tpu_sc as plsc


def _sort_body(v_ref, o_ref):
    x = v_ref[0]
    _, R, LN = x.shape
    L = R * LN
    nbits = L.bit_length() - 1
    rr = lax.broadcasted_iota(jnp.int32, (1, R, LN), 1)
    cc = lax.broadcasted_iota(jnp.int32, (1, R, LN), 2)
    ii = rr * LN + cc

    for p in range(nbits):
        dirmask = 2 << p
        for j in range(p, -1, -1):
            d = 1 << j
            if d < LN:
                up = pltpu.roll(x, LN - d, axis=2)
                dn = pltpu.roll(x, d, axis=2)
            else:
                s = d // LN
                up = jnp.concatenate([x[:, s:, :], x[:, :s, :]], axis=1)
                dn = jnp.concatenate([x[:, R - s:, :], x[:, :R - s, :]],
                                     axis=1)
            is_lower = (ii & d) == 0
            asc = (ii & dirmask) == 0
            part = jnp.where(is_lower, up, dn)
            take_min = asc == is_lower
            x = jnp.where(take_min, jnp.minimum(x, part),
                          jnp.maximum(x, part))
    o_ref[0] = x


def _sort_columns(v):
    B, L, D = v.shape
    LN = min(128, L)
    R = L // LN
    C = 4
    vt = v.transpose(0, 2, 1).reshape(B, D, R, LN)
    st = pl.pallas_call(
        _sort_body,
        grid=(B, D // C),
        in_specs=[pl.BlockSpec((1, C, R, LN), lambda b, c: (b, c, 0, 0))],
        out_specs=pl.BlockSpec((1, C, R, LN), lambda b, c: (b, c, 0, 0)),
        out_shape=jax.ShapeDtypeStruct((B, D, R, LN), jnp.float32),
    )(vt)
    return st.reshape(B, D, L).transpose(0, 2, 1)


def _rank_body(L, cT_ref, ci_ref, rank_ref, cls_ref, acc):
    b = pl.program_id(0)
    i_blk = pl.program_id(1)
    CI = ci_ref.shape[-1]
    cT = cT_ref[...]
    bsel = lax.broadcasted_iota(jnp.int32, cT.shape, 1) == b
    vj = jnp.sum(jnp.where(bsel, cT, 0.0), axis=1, keepdims=True)
    vi = ci_ref[0]
    jj = lax.broadcasted_iota(jnp.int32, (L, 1), 0)
    iidx = i_blk * CI + lax.broadcasted_iota(jnp.int32, (1, CI), 1)
    lt = vj < vi
    eq = (vj == vi) & (jj < iidx)
    cnt = jnp.sum((lt | eq).astype(jnp.int32), axis=0, keepdims=True)
    rank_ref[0] = cnt + b * L

    @pl.when(i_blk == 0)
    def _():
        acc[...] = jnp.zeros_like(acc)

    acc[...] += jnp.where(cnt == 0, iidx, 0)

    @pl.when(i_blk == pl.num_programs(1) - 1)
    def _():
        cls_ref[0] = jnp.broadcast_to(
            jnp.sum(acc[...], axis=1, keepdims=True), cls_ref.shape[1:])


def _rank_col0(col0):
    B, L = col0.shape
    CI = min(128, L)
    NI = L // CI
    col0T = col0.T.reshape(L, B)
    col0i = col0.reshape(B * NI, 1, CI)
    frank, cls3 = pl.pallas_call(
        functools.partial(_rank_body, L),
        grid=(B, NI),
        in_specs=[
            pl.BlockSpec((L, B), lambda b, i: (0, 0)),
            pl.BlockSpec((1, 1, CI), lambda b, i: (b * NI + i, 0, 0)),
        ],
        out_specs=[
            pl.BlockSpec((1, 1, CI), lambda b, i: (b * NI + i, 0, 0)),
            pl.BlockSpec((1, 1, CI), lambda b, i: (b, 0, 0)),
        ],
        out_shape=[
            jax.ShapeDtypeStruct((B * NI, 1, CI), jnp.int32),
            jax.ShapeDtypeStruct((B, 1, CI), jnp.int32),
        ],
        scratch_shapes=[pltpu.VMEM((1, CI), jnp.int32)],
    )(col0T, col0i)
    return frank.reshape(B * L), cls3[:, 0, :1]


def _make_gather(R, D):
    info = plsc.get_sparse_core_info()
    NW = info.num_cores * info.num_subcores
    rows_w = R // NW
    CH = 64
    NCH = rows_w // CH
    mesh = plsc.VectorSubcoreMesh(core_axis_name="c", subcore_axis_name="s")

    @functools.partial(
        pl.kernel,
        out_type=jax.ShapeDtypeStruct((R, D), jnp.float32),
        mesh=mesh,
        scratch_types=[
            pltpu.VMEM((CH,), jnp.int32),
            pltpu.VMEM((CH, D), jnp.float32),
            pltpu.SemaphoreType.DMA,
        ],
    )
    def gather(table_hbm, idx_hbm, out_hbm, idx_v, rows_v, sem):
        wid = lax.axis_index("s") * info.num_cores + lax.axis_index("c")
        base = wid * rows_w

        def chunk(c, carry):
            off = base + c * CH
            pltpu.sync_copy(idx_hbm.at[pl.ds(off, CH)], idx_v)
            pltpu.async_copy(table_hbm.at[idx_v], rows_v, sem).wait()
            pltpu.sync_copy(rows_v, out_hbm.at[pl.ds(off, CH)])
            return carry

        lax.fori_loop(0, NCH, chunk, 0)

    return gather


def kernel(q, k, v):
    del q, k
    B, L, D = v.shape
    frank, cls = _rank_col0(v[:, :, 0])
    sorted_v = _sort_columns(v)
    out = _make_gather(B * L, D)(sorted_v.reshape(B * L, D), frank)
    out = out.reshape(B, L, D)
    return (out, out, cls)
